# Initial kernel scaffold; baseline (speedup 1.0000x reference)
import jax
import jax.numpy as jnp
from jax.experimental import pallas as pl


def kernel(sionna_obj_idx, pos, geo_feat, interaction_type, objW0, objb0, objW1, objb1, objW2, objb2, intW0, intb0, intW1, intb1, intW2, intb2, intW3, intb3, intW4, intb4):
    raise NotImplementedError("write your pallas kernel here")


# speedup vs baseline: 1.3741x; 1.3741x over previous
import functools

import jax
import jax.numpy as jnp
import numpy as np
from jax.experimental import pallas as pl
from jax.experimental.pallas import tpu as pltpu

NUM_OBJECTS = 8
NUM_PARTS = 64
L_ENC = 10
E = NUM_OBJECTS + 1
TILE = 512


def _obj_affine():
    rs = np.random.RandomState(0)
    thetas = rs.uniform(-np.pi, np.pi, size=(NUM_OBJECTS,))
    rots = []
    for th in thetas:
        c, s = np.cos(th), np.sin(th)
        m = np.array([[c, -s, 0.0], [s, c, 0.0], [0.0, 0.0, 1.0]], dtype=np.float64)
        rots.append(np.linalg.inv(m))
    rots.append(np.zeros((3, 3)))
    rot = np.stack(rots, 0).astype(np.float32)
    loc = rs.uniform(-20, 20, size=(NUM_OBJECTS, 3))
    loc = np.concatenate([loc, np.zeros((1, 3))], 0).astype(np.float32)
    half = rs.uniform(0.5, 5.0, size=(NUM_OBJECTS, 3))
    dim = np.stack([-half, half], 1)
    dim = np.concatenate([dim, np.zeros((1, 2, 3))], 0).astype(np.float32)
    scale = np.amax(dim[:, 1, :] - dim[:, 0, :], -1)
    scale[-1] = 1.0
    center = (dim[:, 1, :] + dim[:, 0, :]) / 2.0
    return np.concatenate(
        [rot.reshape(E, 9), loc, center, scale[:, None]], axis=1
    ).astype(np.float32)


_PARAMS = _obj_affine()

_TWOPI = 2.0 * np.pi
_INV2PI = float(np.float32(1.0 / _TWOPI))


def _split9(v):
    f = np.float32(v)
    bits = f.view(np.uint32) & np.uint32(0xFFFF8000)
    return float(bits.view(np.float32))


_TWOPI_1 = _split9(_TWOPI)
_TWOPI_2 = _split9(_TWOPI - _TWOPI_1)
_TWOPI_3 = float(np.float32(_TWOPI - _TWOPI_1 - _TWOPI_2))


def _mlp_kernel(s_ref, x_ref, params_ref,
                w0_ref, b0_ref, w1_ref, b1_ref, w2_ref, b2_ref,
                iw0_ref, ib0_ref, iw1_ref, ib1_ref, iw2_ref, ib2_ref,
                iw3_ref, ib3_ref, iw4_ref, ib4_ref, out_ref):
    f32 = jnp.float32
    dot = functools.partial(jnp.dot, preferred_element_type=f32)
    dot_hi = functools.partial(jnp.dot, preferred_element_type=f32,
                               precision=jax.lax.Precision.HIGHEST)

    s = s_ref[:, :]
    oi = jnp.where(s == -1, NUM_PARTS, s) % NUM_OBJECTS
    oi = jnp.where((s == -1) | (s == NUM_PARTS), NUM_OBJECTS, oi)

    eids = jax.lax.broadcasted_iota(jnp.int32, (TILE, E), 1)
    onehot = (oi == eids).astype(f32)
    pp = dot_hi(onehot, params_ref[:, :])

    d = [x_ref[:, j:j + 1] - pp[:, 9 + j:10 + j] for j in range(3)]
    xu = []
    for j in range(3):
        m = pp[:, 3 * j:3 * j + 1] * d[0] + pp[:, 3 * j + 1:3 * j + 2] * d[1] \
            + pp[:, 3 * j + 2:3 * j + 3] * d[2]
        xu.append((m - pp[:, 12 + j:13 + j]) / pp[:, 15:16] * 2.0)
    xu = jnp.concatenate(xu, axis=-1)

    inv2pi = jnp.float32(_INV2PI)
    c1, c2, c3 = (jnp.float32(_TWOPI_1), jnp.float32(_TWOPI_2),
                  jnp.float32(_TWOPI_3))
    parts = [xu]
    for i in range(L_ENC):
        sc = xu * (2.0 ** i)
        k = jnp.floor(sc * inv2pi + 0.5)
        r = ((sc - k * c1) - k * c2) - k * c3
        parts.append(jnp.sin(r))
        parts.append(jnp.cos(r))
    parts.append(jnp.zeros((TILE, 1), f32))
    enc = jnp.concatenate(parts, axis=-1)

    geo = x_ref[:, 16:32]
    z = jnp.concatenate([jnp.zeros((TILE, 48), f32), geo], axis=-1)
    for e in range(E):
        h = jnp.maximum(dot(enc, w0_ref[e]) + b0_ref[e], 0.0)
        h = jnp.maximum(dot(h, w1_ref[e]) + b1_ref[e], 0.0)
        ze = dot(h, w2_ref[e]) + b2_ref[e]
        z = z + onehot[:, e:e + 1] * ze

    a = jnp.maximum(dot(z, iw0_ref[:, :]) + ib0_ref[:, :], 0.0)
    a = jnp.maximum(dot(a, iw1_ref[:, :]) + ib1_ref[:, :], 0.0)
    a = jnp.maximum(dot(a, iw2_ref[:, :]) + ib2_ref[:, :], 0.0)
    a = jnp.maximum(dot(a, iw3_ref[:, :]) + ib3_ref[:, :], 0.0)
    out_ref[:, :] = dot(a, iw4_ref[:, :]) + ib4_ref[:, :]


def kernel(sionna_obj_idx, pos, geo_feat, interaction_type,
           objW0, objb0, objW1, objb1, objW2, objb2,
           intW0, intb0, intW1, intb1, intW2, intb2, intW3, intb3,
           intW4, intb4):
    data_shape = sionna_obj_idx.shape
    N = int(np.prod(data_shape))
    NT = N // TILE
    f32 = jnp.float32

    s2 = sionna_obj_idx.reshape(N, 1).astype(jnp.int32)
    X = jnp.concatenate(
        [pos.reshape(N, 3).astype(f32),
         jnp.zeros((N, 13), f32),
         geo_feat.reshape(N, 16).astype(f32)], axis=-1)

    params = jnp.asarray(_PARAMS)
    w0p = jnp.concatenate([objW0, jnp.zeros((E, 1, 64), f32)], axis=1)
    w2p = jnp.concatenate([objW2, jnp.zeros((E, 64, 16), f32)], axis=2)
    b2p = jnp.concatenate([objb2, jnp.zeros((E, 16), f32)], axis=1)
    w4p = jnp.concatenate([intW4, jnp.zeros((128, 8), f32)], axis=1)
    b4p = jnp.concatenate([intb4, jnp.zeros((8,), f32)]).reshape(1, 16)

    full = lambda shape: pl.BlockSpec(shape, lambda i: (0,) * len(shape))
    out = pl.pallas_call(
        _mlp_kernel,
        grid=(NT,),
        in_specs=[
            pl.BlockSpec((TILE, 1), lambda i: (i, 0)),
            pl.BlockSpec((TILE, 32), lambda i: (i, 0)),
            full((E, 16)),
            full((E, 64, 64)), full((E, 64)),
            full((E, 64, 64)), full((E, 64)),
            full((E, 64, 64)), full((E, 64)),
            full((64, 128)), full((1, 128)),
            full((128, 128)), full((1, 128)),
            full((128, 128)), full((1, 128)),
            full((128, 128)), full((1, 128)),
            full((128, 16)), full((1, 16)),
        ],
        out_specs=pl.BlockSpec((TILE, 16), lambda i: (i, 0)),
        out_shape=jax.ShapeDtypeStruct((N, 16), f32),
    )(s2, X, params,
      w0p, objb0, objW1, objb1, w2p, b2p,
      intW0, intb0.reshape(1, 128), intW1, intb1.reshape(1, 128),
      intW2, intb2.reshape(1, 128), intW3, intb3.reshape(1, 128),
      w4p, b4p)

    o = out[:, :8]
    tc = jax.lax.complex(o[:, :4], o[:, 4:8]).reshape(data_shape + (4,))
    return jnp.stack(jnp.split(tc, 2, axis=-1), -1)


# speedup vs baseline: 2.8971x; 2.1084x over previous
import functools

import jax
import jax.numpy as jnp
import numpy as np
from jax.experimental import pallas as pl
from jax.experimental.pallas import tpu as pltpu

NUM_OBJECTS = 8
NUM_PARTS = 64
L_ENC = 10
E = NUM_OBJECTS + 1
TILE = 512


def _obj_affine():
    rs = np.random.RandomState(0)
    thetas = rs.uniform(-np.pi, np.pi, size=(NUM_OBJECTS,))
    rots = []
    for th in thetas:
        c, s = np.cos(th), np.sin(th)
        m = np.array([[c, -s, 0.0], [s, c, 0.0], [0.0, 0.0, 1.0]], dtype=np.float64)
        rots.append(np.linalg.inv(m))
    rots.append(np.zeros((3, 3)))
    rot = np.stack(rots, 0).astype(np.float32)
    loc = rs.uniform(-20, 20, size=(NUM_OBJECTS, 3))
    loc = np.concatenate([loc, np.zeros((1, 3))], 0).astype(np.float32)
    half = rs.uniform(0.5, 5.0, size=(NUM_OBJECTS, 3))
    dim = np.stack([-half, half], 1)
    dim = np.concatenate([dim, np.zeros((1, 2, 3))], 0).astype(np.float32)
    scale = np.amax(dim[:, 1, :] - dim[:, 0, :], -1)
    scale[-1] = 1.0
    center = (dim[:, 1, :] + dim[:, 0, :]) / 2.0
    return np.concatenate(
        [rot.reshape(E, 9), loc, center, scale[:, None]], axis=1
    ).astype(np.float32)


_PARAMS = _obj_affine()

_TWOPI = 2.0 * np.pi
_INV2PI = float(np.float32(1.0 / _TWOPI))


def _split9(v):
    f = np.float32(v)
    bits = f.view(np.uint32) & np.uint32(0xFFFF8000)
    return float(bits.view(np.float32))


_TWOPI_1 = _split9(_TWOPI)
_TWOPI_2 = _split9(_TWOPI - _TWOPI_1)
_TWOPI_3 = float(np.float32(_TWOPI - _TWOPI_1 - _TWOPI_2))

_ENC_PERM = np.array(
    [3 + 6 * (f // 3) + f % 3 for f in range(30)]
    + [6 + 6 * (f // 3) + f % 3 for f in range(30)]
    + [0, 1, 2], dtype=np.int32)


def _mlp_kernel(s_ref, x_ref, params_ref,
                w0_ref, b0_ref, w1_ref, b1_ref, w2_ref, b2_ref,
                iw0_ref, ib0_ref, iw1_ref, ib1_ref, iw2_ref, ib2_ref,
                iw3_ref, ib3_ref, iw4_ref, ib4_ref, out_ref):
    f32 = jnp.float32
    dot = functools.partial(jnp.dot, preferred_element_type=f32)
    dot_hi = functools.partial(jnp.dot, preferred_element_type=f32,
                               precision=jax.lax.Precision.HIGHEST)

    s = s_ref[:, :]
    oi = jnp.where(s == -1, NUM_PARTS, s) % NUM_OBJECTS
    oi = jnp.where((s == -1) | (s == NUM_PARTS), NUM_OBJECTS, oi)

    eids = jax.lax.broadcasted_iota(jnp.int32, (TILE, E), 1)
    onehot = (oi == eids).astype(f32)
    pp = dot_hi(onehot, params_ref[:, :])

    d = [x_ref[:, j:j + 1] - pp[:, 9 + j:10 + j] for j in range(3)]
    xu = []
    for j in range(3):
        m = pp[:, 3 * j:3 * j + 1] * d[0] + pp[:, 3 * j + 1:3 * j + 2] * d[1] \
            + pp[:, 3 * j + 2:3 * j + 3] * d[2]
        xu.append((m - pp[:, 12 + j:13 + j]) / pp[:, 15:16] * 2.0)
    xu = jnp.concatenate(xu, axis=-1)

    inv2pi = jnp.float32(_INV2PI)
    c1, c2, c3 = (jnp.float32(_TWOPI_1), jnp.float32(_TWOPI_2),
                  jnp.float32(_TWOPI_3))
    scm = jnp.concatenate([xu * (2.0 ** i) for i in range(L_ENC)], axis=-1)
    k = jnp.floor(scm * inv2pi + 0.5)
    r = ((scm - k * c1) - k * c2) - k * c3
    argm = jnp.concatenate([r, r + jnp.float32(np.pi / 2),
                            jnp.zeros((TILE, 4), f32)], axis=-1)
    sins = jnp.sin(argm)
    enc = jnp.concatenate([sins[:, :60], xu,
                           jnp.zeros((TILE, 1), f32)], axis=-1)

    geo = x_ref[:, 16:32]
    z = jnp.concatenate([jnp.zeros((TILE, 48), f32), geo], axis=-1)
    for e in range(E):
        h = jnp.maximum(dot(enc, w0_ref[e]) + b0_ref[e], 0.0)
        h = jnp.maximum(dot(h, w1_ref[e]) + b1_ref[e], 0.0)
        ze = dot(h, w2_ref[e]) + b2_ref[e]
        z = z + onehot[:, e:e + 1] * ze

    a = jnp.maximum(dot(z, iw0_ref[:, :]) + ib0_ref[:, :], 0.0)
    a = jnp.maximum(dot(a, iw1_ref[:, :]) + ib1_ref[:, :], 0.0)
    a = jnp.maximum(dot(a, iw2_ref[:, :]) + ib2_ref[:, :], 0.0)
    a = jnp.maximum(dot(a, iw3_ref[:, :]) + ib3_ref[:, :], 0.0)
    out_ref[:, :] = dot(a, iw4_ref[:, :]) + ib4_ref[:, :]


def kernel(sionna_obj_idx, pos, geo_feat, interaction_type,
           objW0, objb0, objW1, objb1, objW2, objb2,
           intW0, intb0, intW1, intb1, intW2, intb2, intW3, intb3,
           intW4, intb4):
    data_shape = sionna_obj_idx.shape
    N = int(np.prod(data_shape))
    NT = N // TILE
    f32 = jnp.float32

    s2 = sionna_obj_idx.reshape(N, 1).astype(jnp.int32)
    X = jnp.concatenate(
        [pos.reshape(N, 3).astype(f32),
         jnp.zeros((N, 13), f32),
         geo_feat.reshape(N, 16).astype(f32)], axis=-1)

    params = jnp.asarray(_PARAMS)
    w0p = jnp.concatenate([objW0[:, _ENC_PERM, :],
                           jnp.zeros((E, 1, 64), f32)], axis=1)
    w2p = jnp.concatenate([objW2, jnp.zeros((E, 64, 16), f32)], axis=2)
    b2p = jnp.concatenate([objb2, jnp.zeros((E, 16), f32)], axis=1)
    w4p = jnp.concatenate([intW4, jnp.zeros((128, 8), f32)], axis=1)
    b4p = jnp.concatenate([intb4, jnp.zeros((8,), f32)]).reshape(1, 16)

    full = lambda shape: pl.BlockSpec(shape, lambda i: (0,) * len(shape))
    out = pl.pallas_call(
        _mlp_kernel,
        grid=(NT,),
        in_specs=[
            pl.BlockSpec((TILE, 1), lambda i: (i, 0)),
            pl.BlockSpec((TILE, 32), lambda i: (i, 0)),
            full((E, 16)),
            full((E, 64, 64)), full((E, 64)),
            full((E, 64, 64)), full((E, 64)),
            full((E, 64, 64)), full((E, 64)),
            full((64, 128)), full((1, 128)),
            full((128, 128)), full((1, 128)),
            full((128, 128)), full((1, 128)),
            full((128, 128)), full((1, 128)),
            full((128, 16)), full((1, 16)),
        ],
        out_specs=pl.BlockSpec((TILE, 16), lambda i: (i, 0)),
        out_shape=jax.ShapeDtypeStruct((N, 16), f32),
    )(s2, X, params,
      w0p, objb0, objW1, objb1, w2p, b2p,
      intW0, intb0.reshape(1, 128), intW1, intb1.reshape(1, 128),
      intW2, intb2.reshape(1, 128), intW3, intb3.reshape(1, 128),
      w4p, b4p)

    o = out[:, :8]
    tc = jax.lax.complex(o[:, :4], o[:, 4:8]).reshape(data_shape + (4,))
    return jnp.stack(jnp.split(tc, 2, axis=-1), -1)


# speedup vs baseline: 3.2792x; 1.1319x over previous
import functools

import jax
import jax.numpy as jnp
import numpy as np
from jax.experimental import pallas as pl
from jax.experimental.pallas import tpu as pltpu

NUM_OBJECTS = 8
NUM_PARTS = 64
L_ENC = 10
E = NUM_OBJECTS + 1
TILE = 1024


def _obj_affine():
    rs = np.random.RandomState(0)
    thetas = rs.uniform(-np.pi, np.pi, size=(NUM_OBJECTS,))
    rots = []
    for th in thetas:
        c, s = np.cos(th), np.sin(th)
        m = np.array([[c, -s, 0.0], [s, c, 0.0], [0.0, 0.0, 1.0]], dtype=np.float64)
        rots.append(np.linalg.inv(m))
    rots.append(np.zeros((3, 3)))
    rot = np.stack(rots, 0).astype(np.float32)
    loc = rs.uniform(-20, 20, size=(NUM_OBJECTS, 3))
    loc = np.concatenate([loc, np.zeros((1, 3))], 0).astype(np.float32)
    half = rs.uniform(0.5, 5.0, size=(NUM_OBJECTS, 3))
    dim = np.stack([-half, half], 1)
    dim = np.concatenate([dim, np.zeros((1, 2, 3))], 0).astype(np.float32)
    scale = np.amax(dim[:, 1, :] - dim[:, 0, :], -1)
    scale[-1] = 1.0
    center = (dim[:, 1, :] + dim[:, 0, :]) / 2.0
    return np.concatenate(
        [rot.reshape(E, 9), loc, center, scale[:, None]], axis=1
    ).astype(np.float32)


_PARAMS = _obj_affine()

_TWOPI = 2.0 * np.pi
_INV2PI = float(np.float32(1.0 / _TWOPI))


def _split9(v):
    f = np.float32(v)
    bits = f.view(np.uint32) & np.uint32(0xFFFF8000)
    return float(bits.view(np.float32))


_TWOPI_1 = _split9(_TWOPI)
_TWOPI_2 = _split9(_TWOPI - _TWOPI_1)
_TWOPI_3 = float(np.float32(_TWOPI - _TWOPI_1 - _TWOPI_2))

_ENC_PERM = np.array(
    [3 + 6 * (f // 3) + f % 3 for f in range(30)]
    + [6 + 6 * (f // 3) + f % 3 for f in range(30)]
    + [0, 1, 2], dtype=np.int32)


def _mlp_kernel(s_ref, x_ref, params_ref,
                w0_ref, b0_ref, w1_ref, b1_ref, w2_ref, b2_ref,
                iw0_ref, ib0_ref, iw1_ref, ib1_ref, iw2_ref, ib2_ref,
                iw3_ref, ib3_ref, iw4_ref, ib4_ref, out_ref):
    f32 = jnp.float32
    dot = functools.partial(jnp.dot, preferred_element_type=f32)
    dot_hi = functools.partial(jnp.dot, preferred_element_type=f32,
                               precision=jax.lax.Precision.HIGHEST)

    s = s_ref[:, :]
    oi = jnp.where(s == -1, NUM_PARTS, s) % NUM_OBJECTS
    oi = jnp.where((s == -1) | (s == NUM_PARTS), NUM_OBJECTS, oi)

    eids = jax.lax.broadcasted_iota(jnp.int32, (TILE, E), 1)
    onehot = (oi == eids).astype(f32)
    pp = dot_hi(onehot, params_ref[:, :])

    d = [x_ref[:, j:j + 1] - pp[:, 9 + j:10 + j] for j in range(3)]
    xu = []
    for j in range(3):
        m = pp[:, 3 * j:3 * j + 1] * d[0] + pp[:, 3 * j + 1:3 * j + 2] * d[1] \
            + pp[:, 3 * j + 2:3 * j + 3] * d[2]
        xu.append((m - pp[:, 12 + j:13 + j]) / pp[:, 15:16] * 2.0)
    xu = jnp.concatenate(xu, axis=-1)

    inv2pi = jnp.float32(_INV2PI)
    c1, c2, c3 = (jnp.float32(_TWOPI_1), jnp.float32(_TWOPI_2),
                  jnp.float32(_TWOPI_3))
    scm = jnp.concatenate([xu * (2.0 ** i) for i in range(L_ENC)], axis=-1)
    k = jnp.floor(scm * inv2pi + 0.5)
    r = ((scm - k * c1) - k * c2) - k * c3
    argm = jnp.concatenate([r, r + jnp.float32(np.pi / 2),
                            jnp.zeros((TILE, 4), f32)], axis=-1)
    sins = jnp.sin(argm)
    enc = jnp.concatenate([sins[:, :60], xu,
                           jnp.zeros((TILE, 1), f32)], axis=-1)

    geo = x_ref[:, 16:32]
    z = jnp.concatenate([jnp.zeros((TILE, 48), f32), geo], axis=-1)
    for e in range(E):
        h = jnp.maximum(dot(enc, w0_ref[e]) + b0_ref[e], 0.0)
        h = jnp.maximum(dot(h, w1_ref[e]) + b1_ref[e], 0.0)
        ze = dot(h, w2_ref[e]) + b2_ref[e]
        z = z + onehot[:, e:e + 1] * ze

    a = jnp.maximum(dot(z, iw0_ref[:, :]) + ib0_ref[:, :], 0.0)
    a = jnp.maximum(dot(a, iw1_ref[:, :]) + ib1_ref[:, :], 0.0)
    a = jnp.maximum(dot(a, iw2_ref[:, :]) + ib2_ref[:, :], 0.0)
    a = jnp.maximum(dot(a, iw3_ref[:, :]) + ib3_ref[:, :], 0.0)
    out_ref[:, :] = dot(a, iw4_ref[:, :]) + ib4_ref[:, :]


def kernel(sionna_obj_idx, pos, geo_feat, interaction_type,
           objW0, objb0, objW1, objb1, objW2, objb2,
           intW0, intb0, intW1, intb1, intW2, intb2, intW3, intb3,
           intW4, intb4):
    data_shape = sionna_obj_idx.shape
    N = int(np.prod(data_shape))
    NT = N // TILE
    f32 = jnp.float32

    s2 = sionna_obj_idx.reshape(N, 1).astype(jnp.int32)
    X = jnp.concatenate(
        [pos.reshape(N, 3).astype(f32),
         jnp.zeros((N, 13), f32),
         geo_feat.reshape(N, 16).astype(f32)], axis=-1)

    params = jnp.asarray(_PARAMS)
    w0p = jnp.concatenate([objW0[:, _ENC_PERM, :],
                           jnp.zeros((E, 1, 64), f32)], axis=1)
    w2p = jnp.concatenate([objW2, jnp.zeros((E, 64, 16), f32)], axis=2)
    b2p = jnp.concatenate([objb2, jnp.zeros((E, 16), f32)], axis=1)
    w4p = jnp.concatenate([intW4, jnp.zeros((128, 8), f32)], axis=1)
    b4p = jnp.concatenate([intb4, jnp.zeros((8,), f32)]).reshape(1, 16)

    full = lambda shape: pl.BlockSpec(shape, lambda i: (0,) * len(shape))
    out = pl.pallas_call(
        _mlp_kernel,
        grid=(NT,),
        in_specs=[
            pl.BlockSpec((TILE, 1), lambda i: (i, 0)),
            pl.BlockSpec((TILE, 32), lambda i: (i, 0)),
            full((E, 16)),
            full((E, 64, 64)), full((E, 64)),
            full((E, 64, 64)), full((E, 64)),
            full((E, 64, 64)), full((E, 64)),
            full((64, 128)), full((1, 128)),
            full((128, 128)), full((1, 128)),
            full((128, 128)), full((1, 128)),
            full((128, 128)), full((1, 128)),
            full((128, 16)), full((1, 16)),
        ],
        out_specs=pl.BlockSpec((TILE, 16), lambda i: (i, 0)),
        out_shape=jax.ShapeDtypeStruct((N, 16), f32),
    )(s2, X, params,
      w0p, objb0, objW1, objb1, w2p, b2p,
      intW0, intb0.reshape(1, 128), intW1, intb1.reshape(1, 128),
      intW2, intb2.reshape(1, 128), intW3, intb3.reshape(1, 128),
      w4p, b4p)

    o = out[:, :8]
    tc = jax.lax.complex(o[:, :4], o[:, 4:8]).reshape(data_shape + (4,))
    return jnp.stack(jnp.split(tc, 2, axis=-1), -1)
